# baseline (speedup 1.0000x reference)
import functools

import jax
import jax.numpy as jnp
from jax import lax
from jax.experimental import pallas as pl
from jax.experimental.pallas import tpu as pltpu
from jax.experimental.pallas import tpu_sc as plsc

VOCAB = 1000000
EMBED_DIM = 64
HIDDEN = 64
N_IDX = 16384

NUM_WORKERS = 32
PER_W = N_IDX // NUM_WORKERS
BATCH = 16
NBATCH = PER_W // BATCH
V_BLK = 50000
NB = VOCAB // V_BLK


def _sc_gather_partials(idx3, embeddings):
    mesh = plsc.VectorSubcoreMesh(core_axis_name="c", subcore_axis_name="s")

    @functools.partial(
        pl.kernel,
        mesh=mesh,
        out_type=jax.ShapeDtypeStruct((NUM_WORKERS, EMBED_DIM), jnp.float32),
        scratch_types=[
            pltpu.VMEM((PER_W,), jnp.int32),
            pltpu.VMEM((2, BATCH, EMBED_DIM), jnp.float32),
            pltpu.VMEM((EMBED_DIM,), jnp.float32),
            pltpu.SemaphoreType.DMA,
            pltpu.SemaphoreType.DMA,
        ],
    )
    def k(idx_hbm, emb_hbm, out_hbm, idx_s, rows_v, part_v, sem0, sem1):
        wid = lax.axis_index("s") * 2 + lax.axis_index("c")
        pltpu.sync_copy(idx_hbm.at[wid], idx_s)
        sems = (sem0, sem1)

        def fire(g, buf):
            iv = idx_s[pl.ds(g * BATCH, BATCH)]
            for s in range(BATCH):
                pltpu.async_copy(
                    emb_hbm.at[iv[s]],
                    rows_v.at[buf, s],
                    sems[buf],
                )

        def drain_acc(buf, acc):
            pltpu.make_async_copy(
                emb_hbm.at[pl.ds(0, BATCH)], rows_v.at[buf], sems[buf]
            ).wait()
            for s in range(BATCH):
                acc = tuple(
                    acc[q] + rows_v[buf, s, pl.ds(q * 16, 16)]
                    for q in range(4)
                )
            return acc

        zero = jnp.zeros((16,), jnp.float32)
        fire(0, 0)
        fire(1, 1)

        def body(i, acc):
            g = i * 2
            acc = drain_acc(0, acc)

            @pl.when(g + 2 < NBATCH)
            def _():
                fire(g + 2, 0)

            acc = drain_acc(1, acc)

            @pl.when(g + 3 < NBATCH)
            def _():
                fire(g + 3, 1)

            return acc

        acc = lax.fori_loop(0, NBATCH // 2, body, (zero, zero, zero, zero))

        for q in range(4):
            part_v[pl.ds(q * 16, 16)] = acc[q]
        pltpu.sync_copy(part_v, out_hbm.at[wid])

    return k(idx3, embeddings)


RP = V_BLK // 2


def _tc_logits_body(part_ref, w1_ref, b1_ref, w2_ref, b2_ref,
                    log_ref, stat_ref):
    e = jnp.sum(part_ref[...], axis=0, keepdims=True)
    h = jax.lax.dot_general(e, w1_ref[...], (((1,), (1,)), ((), ())),
                            preferred_element_type=jnp.float32)
    h = jnp.maximum(h + b1_ref[...], 0.0)
    z = jnp.zeros((1, 64), jnp.float32)
    s2 = jnp.concatenate(
        [jnp.concatenate([h, z], axis=1),
         jnp.concatenate([z, h], axis=1)], axis=0)
    logits = jax.lax.dot_general(s2, w2_ref[...], (((1,), (1,)), ((), ())),
                                 preferred_element_type=jnp.float32)
    logits = logits + b2_ref[0]
    log_ref[...] = logits[None]
    blk_max = jnp.max(logits)
    blk_sum = jnp.sum(jnp.exp(logits - blk_max))
    lane = lax.broadcasted_iota(jnp.int32, (1, 8, 128), 2)
    stat_ref[...] = jnp.where(lane == 0, blk_max, jnp.where(lane == 1, blk_sum, 0.0))


def _tc_logits(partials, W1, b1r, w2p, b2p):
    return pl.pallas_call(
        _tc_logits_body,
        grid=(NB,),
        in_specs=[
            pl.BlockSpec((NUM_WORKERS, EMBED_DIM), lambda b: (0, 0)),
            pl.BlockSpec((HIDDEN, EMBED_DIM), lambda b: (0, 0)),
            pl.BlockSpec((1, HIDDEN), lambda b: (0, 0)),
            pl.BlockSpec((RP, 128), lambda b: (b, 0)),
            pl.BlockSpec((1, 2, RP), lambda b: (b, 0, 0)),
        ],
        out_specs=[
            pl.BlockSpec((1, 2, RP), lambda b: (b, 0, 0)),
            pl.BlockSpec((1, 8, 128), lambda b: (b, 0, 0)),
        ],
        out_shape=[
            jax.ShapeDtypeStruct((NB, 2, RP), jnp.float32),
            jax.ShapeDtypeStruct((NB, 8, 128), jnp.float32),
        ],
        compiler_params=pltpu.CompilerParams(
            dimension_semantics=("parallel",),
        ),
    )(partials, W1, b1r, w2p, b2p)


def _tc_sub_body(log_ref, stat_ref, out_ref):
    m = jnp.max(stat_ref[:, 0, 0])
    s = jnp.sum(stat_ref[:, 0, 1] * jnp.exp(stat_ref[:, 0, 0] - m))
    lse = m + jnp.log(s)
    out_ref[...] = log_ref[...] - lse


def _tc_logsoftmax(logits3, stats):
    return pl.pallas_call(
        _tc_sub_body,
        grid=(NB,),
        in_specs=[
            pl.BlockSpec((1, 2, RP), lambda b: (b, 0, 0)),
            pl.BlockSpec((NB, 8, 128), lambda b: (0, 0, 0)),
        ],
        out_specs=pl.BlockSpec((1, 2, RP), lambda b: (b, 0, 0)),
        out_shape=jax.ShapeDtypeStruct((NB, 2, RP), jnp.float32),
        compiler_params=pltpu.CompilerParams(
            dimension_semantics=("parallel",),
        ),
    )(logits3, stats)


def kernel(inputs, embeddings, W1, b1, W2, b2):
    idx3 = inputs.astype(jnp.int32).reshape(NUM_WORKERS, PER_W)
    partials = _sc_gather_partials(idx3, embeddings)
    b1r = b1.reshape(1, HIDDEN)
    w2p = W2.reshape(VOCAB // 2, 128)
    b2p = b2.reshape(NB, RP, 2).transpose(0, 2, 1)
    logits3, stats = _tc_logits(partials, W1, b1r, w2p, b2p)
    out3 = _tc_logsoftmax(logits3, stats)
    return out3.transpose(0, 2, 1).reshape(1, VOCAB)


# speedup vs baseline: 2.1476x; 2.1476x over previous
import functools

import jax
import jax.numpy as jnp
from jax import lax
from jax.experimental import pallas as pl
from jax.experimental.pallas import tpu as pltpu
from jax.experimental.pallas import tpu_sc as plsc

VOCAB = 1000000
EMBED_DIM = 64
HIDDEN = 64
N_IDX = 16384

NUM_WORKERS = 32
PER_W = N_IDX // NUM_WORKERS
BATCH = 16
NBATCH = PER_W // BATCH
V_BLK = 10000
NSTREAM = 4
NBT = VOCAB // V_BLK
NGRP = NBT // NSTREAM


def _sc_gather_partials(idx3, embeddings):
    mesh = plsc.VectorSubcoreMesh(core_axis_name="c", subcore_axis_name="s")

    @functools.partial(
        pl.kernel,
        mesh=mesh,
        out_type=jax.ShapeDtypeStruct((NUM_WORKERS, EMBED_DIM), jnp.float32),
        scratch_types=[
            pltpu.VMEM((PER_W,), jnp.int32),
            pltpu.VMEM((2, BATCH, EMBED_DIM), jnp.float32),
            pltpu.VMEM((EMBED_DIM,), jnp.float32),
            pltpu.SemaphoreType.DMA,
            pltpu.SemaphoreType.DMA,
        ],
    )
    def k(idx_hbm, emb_hbm, out_hbm, idx_s, rows_v, part_v, sem0, sem1):
        wid = lax.axis_index("s") * 2 + lax.axis_index("c")
        pltpu.sync_copy(idx_hbm.at[wid], idx_s)
        sems = (sem0, sem1)

        def fire(g, buf):
            iv = idx_s[pl.ds(g * BATCH, BATCH)]
            for s in range(BATCH):
                pltpu.async_copy(
                    emb_hbm.at[iv[s]],
                    rows_v.at[buf, s],
                    sems[buf],
                )

        def drain_acc(buf, acc):
            pltpu.make_async_copy(
                emb_hbm.at[pl.ds(0, BATCH)], rows_v.at[buf], sems[buf]
            ).wait()
            for s in range(BATCH):
                acc = tuple(
                    acc[q] + rows_v[buf, s, pl.ds(q * 16, 16)]
                    for q in range(4)
                )
            return acc

        zero = jnp.zeros((16,), jnp.float32)
        fire(0, 0)
        fire(1, 1)

        def body(i, acc):
            g = i * 2
            acc = drain_acc(0, acc)

            @pl.when(g + 2 < NBATCH)
            def _():
                fire(g + 2, 0)

            acc = drain_acc(1, acc)

            @pl.when(g + 3 < NBATCH)
            def _():
                fire(g + 3, 1)

            return acc

        acc = lax.fori_loop(0, NBATCH // 2, body, (zero, zero, zero, zero))

        for q in range(4):
            part_v[pl.ds(q * 16, 16)] = acc[q]
        pltpu.sync_copy(part_v, out_hbm.at[wid])

    return k(idx3, embeddings)


def _tc_logits_body(part_ref, w1_ref, b1_ref,
                    w2_0, w2_1, w2_2, w2_3,
                    b2_0, b2_1, b2_2, b2_3,
                    log_0, log_1, log_2, log_3,
                    stat_0, stat_1, stat_2, stat_3):
    e = jnp.sum(part_ref[...], axis=0, keepdims=True)
    h = jax.lax.dot_general(e, w1_ref[...], (((1,), (1,)), ((), ())),
                            preferred_element_type=jnp.float32)
    h = jnp.maximum(h + b1_ref[...], 0.0)
    lane = lax.broadcasted_iota(jnp.int32, (1, 8, 128), 2)
    for w2_ref, b2_ref, log_ref, stat_ref in (
            (w2_0, b2_0, log_0, stat_0),
            (w2_1, b2_1, log_1, stat_1),
            (w2_2, b2_2, log_2, stat_2),
            (w2_3, b2_3, log_3, stat_3)):
        logits = jax.lax.dot_general(
            h, w2_ref[...], (((1,), (1,)), ((), ())),
            preferred_element_type=jnp.float32)
        logits = logits + b2_ref[0]
        log_ref[...] = logits[None]
        blk_max = jnp.max(logits)
        blk_sum = jnp.sum(jnp.exp(logits - blk_max))
        stat_ref[...] = jnp.where(lane == 0, blk_max,
                                  jnp.where(lane == 1, blk_sum, 0.0))


def _tc_logits(partials, W1, b1r, W2, b2r):
    w2_specs = [
        pl.BlockSpec((V_BLK, HIDDEN), (lambda b, j=j: (NSTREAM * b + j, 0)))
        for j in range(NSTREAM)
    ]
    b2_specs = [
        pl.BlockSpec((1, 1, V_BLK), (lambda b, j=j: (NSTREAM * b + j, 0, 0)))
        for j in range(NSTREAM)
    ]
    out_specs = (
        [pl.BlockSpec((1, 1, V_BLK), lambda b: (b, 0, 0))] * NSTREAM
        + [pl.BlockSpec((1, 8, 128), lambda b: (b, 0, 0))] * NSTREAM
    )
    out_shape = (
        [jax.ShapeDtypeStruct((NGRP, 1, V_BLK), jnp.float32)] * NSTREAM
        + [jax.ShapeDtypeStruct((NGRP, 8, 128), jnp.float32)] * NSTREAM
    )
    return pl.pallas_call(
        _tc_logits_body,
        grid=(NGRP,),
        in_specs=(
            [
                pl.BlockSpec((NUM_WORKERS, EMBED_DIM), lambda b: (0, 0)),
                pl.BlockSpec((HIDDEN, EMBED_DIM), lambda b: (0, 0)),
                pl.BlockSpec((1, HIDDEN), lambda b: (0, 0)),
            ]
            + w2_specs
            + b2_specs
        ),
        out_specs=out_specs,
        out_shape=out_shape,
        compiler_params=pltpu.CompilerParams(
            dimension_semantics=("parallel",),
        ),
    )(partials, W1, b1r, W2, W2, W2, W2, b2r, b2r, b2r, b2r)


def _tc_sub_body(log_0, log_1, log_2, log_3,
                 stat_0, stat_1, stat_2, stat_3, out_ref):
    mx = jnp.stack([s[:, 0, 0] for s in (stat_0, stat_1, stat_2, stat_3)])
    sm = jnp.stack([s[:, 0, 1] for s in (stat_0, stat_1, stat_2, stat_3)])
    m = jnp.max(mx)
    s = jnp.sum(sm * jnp.exp(mx - m))
    lse = m + jnp.log(s)
    out_ref[...] = jnp.concatenate(
        [log_0[...], log_1[...], log_2[...], log_3[...]], axis=1) - lse


def _tc_logsoftmax(logs, stats):
    log_specs = [pl.BlockSpec((1, 1, V_BLK), lambda b: (b, 0, 0))
                 for _ in range(NSTREAM)]
    stat_specs = [pl.BlockSpec((NGRP, 8, 128), lambda b: (0, 0, 0))
                  for _ in range(NSTREAM)]
    return pl.pallas_call(
        _tc_sub_body,
        grid=(NGRP,),
        in_specs=log_specs + stat_specs,
        out_specs=pl.BlockSpec((1, NSTREAM, V_BLK), lambda b: (b, 0, 0)),
        out_shape=jax.ShapeDtypeStruct((NGRP, NSTREAM, V_BLK), jnp.float32),
        compiler_params=pltpu.CompilerParams(
            dimension_semantics=("parallel",),
        ),
    )(*logs, *stats)


def kernel(inputs, embeddings, W1, b1, W2, b2):
    idx3 = inputs.astype(jnp.int32).reshape(NUM_WORKERS, PER_W)
    partials = _sc_gather_partials(idx3, embeddings)
    b1r = b1.reshape(1, HIDDEN)
    b2r = b2.reshape(NBT, 1, V_BLK)
    outs = _tc_logits(partials, W1, b1r, W2, b2r)
    logs, stats = outs[:NSTREAM], outs[NSTREAM:]
    out3 = _tc_logsoftmax(logs, stats)
    return out3.reshape(1, VOCAB)
